# baseline (speedup 1.0000x reference)
import functools
import jax
import jax.numpy as jnp
from jax import lax
from jax.experimental import pallas as pl
from jax.experimental.pallas import tpu as pltpu
from jax.experimental.pallas import tpu_sc as plsc

N = 10000
E = 320000
D = 128
NP = 10240
NC, NS = 2, 16
NT = NC * NS
C = 128
CH0, CH1 = 80, 80
NCHT = NS * (CH0 + CH1)
EP = NCHT * C
RPT = NP // NS


def _sc_aggregate(feat, src2, dst2, zeros, with_cnt):
    mesh = plsc.VectorSubcoreMesh(core_axis_name="c", subcore_axis_name="s",
                                  num_cores=NC, num_subcores=NS)
    out_type = [jax.ShapeDtypeStruct((NC, NP, D), jnp.float32)]
    scratch = [
        pltpu.VMEM((4, C), jnp.int32),
        pltpu.VMEM((4, C), jnp.int32),
        pltpu.VMEM((C, D), jnp.float32),
        pltpu.VMEM((C, D), jnp.float32),
    ] + [pltpu.SemaphoreType.DMA] * 8 + [
        pltpu.VMEM_SHARED((NP, D), jnp.float32),
    ]
    if with_cnt:
        out_type.append(jax.ShapeDtypeStruct((NT, NP), jnp.float32))
        scratch.append(pltpu.VMEM((NP,), jnp.float32))

    @functools.partial(
        pl.kernel, mesh=mesh, out_type=out_type, scratch_types=scratch,
        compiler_params=pltpu.CompilerParams(needs_layout_passes=False),
    )
    def k(feat_h, src_h, dst_h, z_h, out_h, *rest):
        if with_cnt:
            cnt_h, sidx, didx, r0b, r1b, g0, g1, s0, s1, i0, i1, i2, i3, \
                acc, cntv = rest
        else:
            sidx, didx, r0b, r1b, g0, g1, s0, s1, i0, i1, i2, i3, acc = rest
        rows = [r0b, r1b]
        gs = [g0, g1]
        ss = [s0, s1]
        isem = [i0, i1, i2, i3]
        c = lax.axis_index("c")
        s = lax.axis_index("s")
        r0 = s * RPT
        pltpu.sync_copy(z_h.at[pl.ds(r0, RPT)], acc.at[pl.ds(r0, RPT)])
        if with_cnt:
            def zc(i, carry):
                cntv[pl.ds(i * 16, 16)] = jnp.zeros((16,), jnp.float32)
                return carry
            lax.fori_loop(0, NP // 16, zc, 0)
        plsc.subcore_barrier()

        def run(ebase, nch):
            def si_d(ch, ib):
                pltpu.async_copy(src_h.at[ebase + ch], sidx.at[ib], isem[ib])
                pltpu.async_copy(dst_h.at[ebase + ch], didx.at[ib], isem[ib])

            def wi(ch, ib):
                pltpu.make_async_copy(src_h.at[ebase + ch], sidx.at[ib],
                                      isem[ib]).wait()
                pltpu.make_async_copy(dst_h.at[ebase + ch], didx.at[ib],
                                      isem[ib]).wait()

            def sg(ib, b):
                pltpu.async_copy(feat_h.at[sidx.at[ib]], rows[b], gs[b])

            def wg(ib, b):
                pltpu.make_async_copy(feat_h.at[sidx.at[ib]], rows[b],
                                      gs[b]).wait()

            def sc_(ib, b):
                pltpu.async_copy(rows[b], acc.at[didx.at[ib]], ss[b],
                                 add=True)

            def ws(ib, b):
                pltpu.make_async_copy(rows[b], acc.at[didx.at[ib]],
                                      ss[b]).wait()

            def cu(ib):
                if with_cnt:
                    ones = jnp.ones((16,), jnp.float32)
                    for g in range(C // 16):
                        idx16 = didx[ib, pl.ds(g * 16, 16)]
                        plsc.addupdate_scatter(cntv, [idx16], ones)

            si_d(0, 0); si_d(1, 1); si_d(2, 2)
            wi(0, 0)
            sg(0, 0)
            wi(1, 1)
            wg(0, 0); sc_(0, 0); sg(1, 1); si_d(3, 3); cu(0)
            wg(1, 1); sc_(1, 1); ws(0, 0); wi(2, 2); sg(2, 0)
            si_d(4, 0); cu(1)

            def quad(q, carry):
                base = q * 4 + 2
                for b in range(4):
                    ch = base + b
                    ib = (2 + b) % 4
                    rb = b % 2
                    wg(ib, rb)
                    sc_(ib, rb)
                    ws((ib - 1) % 4, 1 - rb)
                    wi(ch + 1, (ib + 1) % 4)
                    sg((ib + 1) % 4, 1 - rb)
                    si_d(ch + 3, (ib + 3) % 4)
                    cu(ib)
                return carry

            lax.fori_loop(0, (nch - 8) // 4, quad, 0)

            for ch in range(nch - 6, nch):
                ib = ch % 4
                rb = ch % 2
                wg(ib, rb)
                sc_(ib, rb)
                ws((ib - 1) % 4, 1 - rb)
                if ch + 1 < nch:
                    wi(ch + 1, (ch + 1) % 4)
                    sg((ch + 1) % 4, 1 - rb)
                if ch + 3 < nch:
                    si_d(ch + 3, (ch + 3) % 4)
                cu(ib)
            ws((nch - 1) % 4, (nch - 1) % 2)

        @pl.when(c == 0)
        def _():
            run(s * CH0, CH0)

        if CH1:
            @pl.when(c == 1)
            def _():
                run(NS * CH0 + s * CH1, CH1)

        plsc.subcore_barrier()
        pltpu.sync_copy(acc.at[pl.ds(r0, RPT)],
                        out_h.at[c].at[pl.ds(r0, RPT)])
        if with_cnt:
            pltpu.sync_copy(cntv, cnt_h.at[c * NS + s])

    res = k(feat, src2, dst2, zeros)
    if with_cnt:
        return res[0], res[1]
    return res[0] if isinstance(res, (list, tuple)) else res


def _tc_dense(p, cnt, root, WlT, bl, WrT, relu):
    B = 400

    def body(p0_r, p1_r, c_r, x_r, wl_r, bl_r, wr_r, o_r):
        ssum = p0_r[0] + p1_r[0]
        cs = jnp.sum(c_r[...], axis=1, keepdims=True)
        mean = ssum / jnp.maximum(cs, 1.0)
        h = (jnp.dot(mean, wl_r[...], preferred_element_type=jnp.float32)
             + jnp.dot(x_r[...], wr_r[...],
                       preferred_element_type=jnp.float32)
             + bl_r[...])
        if relu:
            h = jnp.maximum(h, 0.0)
        o_r[...] = h

    return pl.pallas_call(
        body,
        grid=(N // B,),
        in_specs=[
            pl.BlockSpec((1, B, D), lambda i: (0, i, 0)),
            pl.BlockSpec((1, B, D), lambda i: (1, i, 0)),
            pl.BlockSpec((B, NT), lambda i: (i, 0)),
            pl.BlockSpec((B, D), lambda i: (i, 0)),
            pl.BlockSpec((D, D), lambda i: (0, 0)),
            pl.BlockSpec((1, D), lambda i: (0, 0)),
            pl.BlockSpec((D, D), lambda i: (0, 0)),
        ],
        out_specs=pl.BlockSpec((B, D), lambda i: (i, 0)),
        out_shape=jax.ShapeDtypeStruct((N, D), jnp.float32),
    )(p, p, cnt, root, WlT, bl, WrT)


def kernel(x, edge_index, W1l, b1l, W1r, W2l, b2l, W2r):
    spad = jnp.zeros((EP - E,), jnp.int32)
    dpad = N + (jnp.arange(EP - E, dtype=jnp.int32) % (NP - N))
    src = jnp.concatenate([edge_index[0], spad]).reshape(EP // C, C)
    dst = jnp.concatenate([edge_index[1], dpad]).reshape(EP // C, C)
    zeros = jnp.zeros((NP, D), jnp.float32)

    p, cnt = _sc_aggregate(x, src, dst, zeros, True)
    cntT = cnt.T
    h = _tc_dense(p, cntT, x, W1l.T, b1l[None, :], W1r.T, True)
    q = _sc_aggregate(h, src, dst, zeros, False)
    return _tc_dense(q, cntT, h, W2l.T, b2l[None, :], W2r.T, False)


# speedup vs baseline: 3.0587x; 3.0587x over previous
import functools
import jax
import jax.numpy as jnp
from jax import lax
from jax.experimental import pallas as pl
from jax.experimental.pallas import tpu as pltpu
from jax.experimental.pallas import tpu_sc as plsc

N = 10000
E = 320000
D = 128
NP = 10240
NC, NS = 2, 16
NT = NC * NS
C = 128
CH0, CH1 = 80, 80
NCHT = NS * (CH0 + CH1)
EP = NCHT * C
RPT = NP // NS


def _sc_aggregate(feat, src2, dst2, zeros, with_cnt):
    mesh = plsc.VectorSubcoreMesh(core_axis_name="c", subcore_axis_name="s",
                                  num_cores=NC, num_subcores=NS)
    out_type = [jax.ShapeDtypeStruct((NC, NP, D), jnp.float32)]
    scratch = [
        pltpu.VMEM((4, C), jnp.int32),
        pltpu.VMEM((4, C), jnp.int32),
        pltpu.VMEM((C, D), jnp.float32),
        pltpu.VMEM((C, D), jnp.float32),
    ] + [pltpu.SemaphoreType.DMA] * 8 + [
        pltpu.VMEM_SHARED((NP, D), jnp.float32),
    ]
    if with_cnt:
        out_type.append(jax.ShapeDtypeStruct((NT, NP), jnp.float32))
        scratch.append(pltpu.VMEM((NP,), jnp.float32))

    @functools.partial(
        pl.kernel, mesh=mesh, out_type=out_type, scratch_types=scratch,
        compiler_params=pltpu.CompilerParams(needs_layout_passes=False),
    )
    def k(feat_h, src_h, dst_h, z_h, out_h, *rest):
        if with_cnt:
            cnt_h, sidx, didx, r0b, r1b, g0, g1, s0, s1, i0, i1, i2, i3, \
                acc, cntv = rest
        else:
            sidx, didx, r0b, r1b, g0, g1, s0, s1, i0, i1, i2, i3, acc = rest
        rows = [r0b, r1b]
        gs = [g0, g1]
        ss = [s0, s1]
        isem = [i0, i1, i2, i3]
        c = lax.axis_index("c")
        s = lax.axis_index("s")
        r0 = s * RPT
        pltpu.sync_copy(z_h.at[pl.ds(r0, RPT)], acc.at[pl.ds(r0, RPT)])
        if with_cnt:
            def zc(i, carry):
                cntv[pl.ds(i * 16, 16)] = jnp.zeros((16,), jnp.float32)
                return carry
            lax.fori_loop(0, NP // 16, zc, 0)
        plsc.subcore_barrier()

        def run(ebase, nch):
            def si_d(ch, ib):
                pltpu.async_copy(src_h.at[ebase + ch], sidx.at[ib], isem[ib])
                pltpu.async_copy(dst_h.at[ebase + ch], didx.at[ib], isem[ib])

            def wi(ch, ib):
                pltpu.make_async_copy(src_h.at[ebase + ch], sidx.at[ib],
                                      isem[ib]).wait()
                pltpu.make_async_copy(dst_h.at[ebase + ch], didx.at[ib],
                                      isem[ib]).wait()

            def sg(ib, b):
                pltpu.async_copy(feat_h.at[sidx.at[ib]], rows[b], gs[b])

            def wg(ib, b):
                pltpu.make_async_copy(feat_h.at[sidx.at[ib]], rows[b],
                                      gs[b]).wait()

            def sc_(ib, b):
                pltpu.async_copy(rows[b], acc.at[didx.at[ib]], ss[b],
                                 add=True)

            def ws(ib, b):
                pltpu.make_async_copy(rows[b], acc.at[didx.at[ib]],
                                      ss[b]).wait()

            def cu(ib):
                if with_cnt:
                    ones = jnp.ones((16,), jnp.float32)
                    for g in range(C // 16):
                        idx16 = didx[ib, pl.ds(g * 16, 16)]
                        plsc.addupdate_scatter(cntv, [idx16], ones)

            si_d(0, 0); si_d(1, 1); si_d(2, 2)
            wi(0, 0)
            sg(0, 0)
            wi(1, 1)
            wg(0, 0); sc_(0, 0); sg(1, 1); si_d(3, 3); cu(0)
            wg(1, 1); sc_(1, 1); ws(0, 0); wi(2, 2); sg(2, 0)
            si_d(4, 0); cu(1)

            def quad(q, carry):
                base = q * 4 + 2
                for b in range(4):
                    ch = base + b
                    ib = (2 + b) % 4
                    rb = b % 2
                    wg(ib, rb)
                    sc_(ib, rb)
                    ws((ib - 1) % 4, 1 - rb)
                    wi(ch + 1, (ib + 1) % 4)
                    sg((ib + 1) % 4, 1 - rb)
                    si_d(ch + 3, (ib + 3) % 4)
                    cu(ib)
                return carry

            lax.fori_loop(0, (nch - 8) // 4, quad, 0)

            for ch in range(nch - 6, nch):
                ib = ch % 4
                rb = ch % 2
                wg(ib, rb)
                sc_(ib, rb)
                ws((ib - 1) % 4, 1 - rb)
                if ch + 1 < nch:
                    wi(ch + 1, (ch + 1) % 4)
                    sg((ch + 1) % 4, 1 - rb)
                if ch + 3 < nch:
                    si_d(ch + 3, (ch + 3) % 4)
                cu(ib)
            ws((nch - 1) % 4, (nch - 1) % 2)

        @pl.when(c == 0)
        def _():
            run(s * CH0, CH0)

        if CH1:
            @pl.when(c == 1)
            def _():
                run(NS * CH0 + s * CH1, CH1)

        plsc.subcore_barrier()
        pltpu.sync_copy(acc.at[pl.ds(r0, RPT)],
                        out_h.at[c].at[pl.ds(r0, RPT)])
        if with_cnt:
            pltpu.sync_copy(cntv, cnt_h.at[c * NS + s])

    res = k(feat, src2, dst2, zeros)
    if with_cnt:
        return res[0], res[1]
    return res[0] if isinstance(res, (list, tuple)) else res


def _tc_dense(p, cnt, root, WlT, bl, WrT, relu):
    B = 400

    def body(p0_r, p1_r, c_r, x_r, wl_r, bl_r, wr_r, o_r):
        ssum = p0_r[0] + p1_r[0]
        cs = jnp.sum(c_r[...], axis=1, keepdims=True)
        mean = ssum / jnp.maximum(cs, 1.0)
        h = (jnp.dot(mean, wl_r[...], preferred_element_type=jnp.float32)
             + jnp.dot(x_r[...], wr_r[...],
                       preferred_element_type=jnp.float32)
             + bl_r[...])
        if relu:
            h = jnp.maximum(h, 0.0)
        o_r[...] = h

    return pl.pallas_call(
        body,
        grid=(N // B,),
        in_specs=[
            pl.BlockSpec((1, B, D), lambda i: (0, i, 0)),
            pl.BlockSpec((1, B, D), lambda i: (1, i, 0)),
            pl.BlockSpec((B, NT), lambda i: (i, 0)),
            pl.BlockSpec((B, D), lambda i: (i, 0)),
            pl.BlockSpec((D, D), lambda i: (0, 0)),
            pl.BlockSpec((1, D), lambda i: (0, 0)),
            pl.BlockSpec((D, D), lambda i: (0, 0)),
        ],
        out_specs=pl.BlockSpec((B, D), lambda i: (i, 0)),
        out_shape=jax.ShapeDtypeStruct((N, D), jnp.float32),
    )(p, p, cnt, root, WlT, bl, WrT)


def kernel(x, edge_index, W1l, b1l, W1r, W2l, b2l, W2r):
    spad = jnp.arange(EP - E, dtype=jnp.int32) % N
    dpad = N + (jnp.arange(EP - E, dtype=jnp.int32) % (NP - N))
    src = jnp.concatenate([edge_index[0], spad]).reshape(EP // C, C)
    dst = jnp.concatenate([edge_index[1], dpad]).reshape(EP // C, C)
    zeros = jnp.zeros((NP, D), jnp.float32)

    p, cnt = _sc_aggregate(x, src, dst, zeros, True)
    cntT = cnt.T
    h = _tc_dense(p, cntT, x, W1l.T, b1l[None, :], W1r.T, True)
    q = _sc_aggregate(h, src, dst, zeros, False)
    return _tc_dense(q, cntT, h, W2l.T, b2l[None, :], W2r.T, False)
